# Initial kernel scaffold; baseline (speedup 1.0000x reference)
import jax
import jax.numpy as jnp
from jax.experimental import pallas as pl


def kernel(x, edge_index, edge_weight, prev_hidden_state, c, Wz_c, bz_c, Wr_c, br_c, Wh_c, bh_c, Wz, bz, Wr, br, Wh, bh, Wlin, blin):
    raise NotImplementedError("write your pallas kernel here")


# speedup vs baseline: 16.0269x; 16.0269x over previous
import functools

import jax
import jax.numpy as jnp
from jax import lax
from jax.experimental import pallas as pl
from jax.experimental.pallas import tpu as pltpu
from jax.experimental.pallas import tpu_sc as plsc

N = 10000
F_IN = 128
F_OUT = 32
F3 = 3 * F_OUT

NC = 2
NS = 16
NW = NC * NS

ROW = 128
RCH = 4
CE = ROW * RCH
FV = F3 // 16


def _rsqrt_newton(d):
    i = plsc.bitcast(d, jnp.int32)
    y = plsc.bitcast(jnp.int32(0x5F3759DF) - (i >> 1), jnp.float32)
    for _ in range(3):
        y = y * (1.5 - 0.5 * d * y * y)
    return y


def _sc_aggregate(src2d, dst2d, w2d, h, n_rows, rows_per_worker):
    rows_per_tile_deg = n_rows // NS
    deg_chunks = rows_per_tile_deg // RCH
    msg_chunks = rows_per_worker // RCH
    npt = N // NS

    mesh = plsc.VectorSubcoreMesh(core_axis_name="c", subcore_axis_name="s")

    @functools.partial(
        pl.kernel,
        out_type=[
            jax.ShapeDtypeStruct((NC, N, F3), jnp.float32),
            jax.ShapeDtypeStruct((N,), jnp.float32),
        ],
        mesh=mesh,
        scratch_types=[
            pltpu.VMEM((N,), jnp.float32),
            pltpu.VMEM((1000,), jnp.float32),
            pltpu.VMEM((RCH, ROW), jnp.int32),
            pltpu.VMEM((RCH, ROW), jnp.int32),
            pltpu.VMEM((RCH, ROW), jnp.float32),
            pltpu.VMEM((CE,), jnp.float32),
            pltpu.VMEM((CE, F3), jnp.float32),
            pltpu.VMEM_SHARED((N,), jnp.float32),
            pltpu.VMEM_SHARED((N, F3), jnp.float32),
            pltpu.SemaphoreType.DMA,
        ],
        compiler_params=pltpu.CompilerParams(
            use_tc_tiling_on_sc=False, needs_layout_passes=False),
    )
    def kern(src_hbm, dst_hbm, w_hbm, h_hbm, acc_out, dinv_out,
             deg_l, zeros_l, srcb, dstb, wb, ab, rows, deg_sh, acc_sh, sem):
        c = lax.axis_index("c")
        s = lax.axis_index("s")
        z16 = jnp.zeros((16,), jnp.float32)

        @pl.loop(0, 1000 // 16)
        def _(i):
            zeros_l[pl.ds(i * 16, 16)] = z16

        @pl.loop(0, CE)
        def _(e):
            for f in range(FV):
                rows[e, pl.ds(f * 16, 16)] = z16

        @pl.when(s < 10)
        def _():
            pltpu.sync_copy(zeros_l, deg_sh.at[pl.ds(s * 1000, 1000)])

        pltpu.sync_copy(rows, acc_sh.at[pl.ds(s * npt, CE)])
        pltpu.sync_copy(rows.at[pl.ds(0, npt - CE)],
                        acc_sh.at[pl.ds(s * npt + CE, npt - CE)])
        plsc.subcore_barrier()

        @pl.loop(0, deg_chunks)
        def _(k):
            r0 = s * rows_per_tile_deg + k * RCH
            pltpu.sync_copy(dst_hbm.at[pl.ds(r0, RCH)], dstb)
            pltpu.sync_copy(w_hbm.at[pl.ds(r0, RCH)], wb)
            for j in range(RCH):
                pltpu.sync_copy(wb.at[j], deg_sh.at[dstb.at[j]], add=True)

        plsc.subcore_barrier()

        pltpu.sync_copy(deg_sh, deg_l)

        @pl.loop(0, N // 16)
        def _(i):
            sl = pl.ds(i * 16, 16)
            deg_l[sl] = _rsqrt_newton(deg_l[sl] + 1.0)

        @pl.when(jnp.logical_and(c == 0, s < 10))
        def _():
            sl = pl.ds(s * 1000, 1000)
            pltpu.sync_copy(deg_l.at[sl], dinv_out.at[sl])

        w0 = (c * NS + s) * rows_per_worker

        @pl.loop(0, msg_chunks)
        def _(k):
            r0 = w0 + k * RCH
            pltpu.sync_copy(src_hbm.at[pl.ds(r0, RCH)], srcb)
            pltpu.sync_copy(dst_hbm.at[pl.ds(r0, RCH)], dstb)
            pltpu.sync_copy(w_hbm.at[pl.ds(r0, RCH)], wb)

            for j in range(RCH):
                pltpu.async_copy(
                    h_hbm.at[srcb.at[j]],
                    rows.at[pl.ds(j * ROW, ROW)], sem).wait()

            for j in range(RCH):
                for i in range(ROW // 16):
                    idx = srcb[j, pl.ds(i * 16, 16)]
                    dv = plsc.load_gather(deg_l, [idx])
                    ab[pl.ds(j * ROW + i * 16, 16)] = \
                        wb[j, pl.ds(i * 16, 16)] * dv

            @pl.loop(0, CE // 16)
            def _(g):
                av = ab[pl.ds(g * 16, 16)]
                e0 = g * 16
                for j in range(16):
                    a = av[j]
                    for f in range(FV):
                        sl = pl.ds(f * 16, 16)
                        rows[e0 + j, sl] = rows[e0 + j, sl] * a

            for j in range(RCH):
                pltpu.sync_copy(rows.at[pl.ds(j * ROW, ROW)],
                                acc_sh.at[dstb.at[j]], add=True)

        plsc.subcore_barrier()

        pltpu.sync_copy(acc_sh.at[pl.ds(s * npt, npt)],
                        acc_out.at[c].at[pl.ds(s * npt, npt)])

    return kern(src2d, dst2d, w2d, h)


def _tc_project(x, wcat):
    nb = 5
    bs = N // nb

    def body(x_ref, w_ref, o_ref):
        o_ref[...] = jnp.dot(x_ref[...], w_ref[...],
                             preferred_element_type=jnp.float32)

    return pl.pallas_call(
        body,
        grid=(nb,),
        in_specs=[
            pl.BlockSpec((bs, F_IN), lambda i: (i, 0)),
            pl.BlockSpec((F_IN, F3), lambda i: (0, 0)),
        ],
        out_specs=pl.BlockSpec((bs, F3), lambda i: (i, 0)),
        out_shape=jax.ShapeDtypeStruct((N, F3), jnp.float32),
    )(x, wcat)


def _tc_gru(acc, h, dinv, hprev, wza, wzb, cz, wra, wrb, cr, wha, whb, ch,
            wlin, blin):
    nb = 5
    bs = N // nb

    def body(a0_ref, a1_ref, h_ref, di_ref, hp_ref, wza_ref, wzb_ref, cz_ref,
             wra_ref, wrb_ref, cr_ref, wha_ref, whb_ref, ch_ref,
             wlin_ref, blin_ref, y_ref, hn_ref):
        di = di_ref[...]
        hp = hp_ref[...]
        agg = (a0_ref[...] + a1_ref[...] + h_ref[...] * di) * di
        gz = agg[:, :F_OUT]
        gr = agg[:, F_OUT:2 * F_OUT]
        gh = agg[:, 2 * F_OUT:]
        f32 = jnp.float32
        z = jax.nn.sigmoid(jnp.dot(gz, wza_ref[...], preferred_element_type=f32)
                           + jnp.dot(hp, wzb_ref[...], preferred_element_type=f32)
                           + cz_ref[...])
        r = jax.nn.sigmoid(jnp.dot(gr, wra_ref[...], preferred_element_type=f32)
                           + jnp.dot(hp, wrb_ref[...], preferred_element_type=f32)
                           + cr_ref[...])
        ht = jnp.tanh(jnp.dot(gh, wha_ref[...], preferred_element_type=f32)
                      + jnp.dot(hp * r, whb_ref[...], preferred_element_type=f32)
                      + ch_ref[...])
        hn = z * hp + (1.0 - z) * ht
        hn_ref[...] = hn
        y_ref[...] = (jnp.dot(jax.nn.relu(hn), wlin_ref[...],
                              preferred_element_type=f32) + blin_ref[...])

    full = lambda r, c: pl.BlockSpec((r, c), lambda i: (0, 0))
    blk = lambda cdim: pl.BlockSpec((bs, cdim), lambda i: (i, 0))
    return pl.pallas_call(
        body,
        grid=(nb,),
        in_specs=[
            blk(F3), blk(F3), blk(F3), blk(1), blk(F_OUT),
            full(F_OUT, F_OUT), full(F_OUT, F_OUT), full(1, F_OUT),
            full(F_OUT, F_OUT), full(F_OUT, F_OUT), full(1, F_OUT),
            full(F_OUT, F_OUT), full(F_OUT, F_OUT), full(1, F_OUT),
            full(F_OUT, 1), full(1, 1),
        ],
        out_specs=[blk(1), blk(F_OUT)],
        out_shape=[
            jax.ShapeDtypeStruct((N, 1), jnp.float32),
            jax.ShapeDtypeStruct((N, F_OUT), jnp.float32),
        ],
    )(acc[0], acc[1], h, dinv, hprev, wza, wzb, cz, wra, wrb, cr,
      wha, whb, ch, wlin, blin)


def kernel(x, edge_index, edge_weight, prev_hidden_state, c,
           Wz_c, bz_c, Wr_c, br_c, Wh_c, bh_c,
           Wz, bz, Wr, br, Wh, bh, Wlin, blin):
    src, dst = edge_index[0], edge_index[1]
    e = src.shape[0]

    epad = -(-e // (NW * CE)) * (NW * CE)
    pad = epad - e
    if pad:
        src = jnp.concatenate([src, jnp.zeros((pad,), src.dtype)])
        dst = jnp.concatenate([dst, jnp.zeros((pad,), dst.dtype)])
        edge_weight = jnp.concatenate(
            [edge_weight, jnp.zeros((pad,), edge_weight.dtype)])
    n_rows = epad // ROW
    src2d = src.reshape(n_rows, ROW)
    dst2d = dst.reshape(n_rows, ROW)
    w2d = edge_weight.reshape(n_rows, ROW)

    wcat = jnp.concatenate([Wz_c, Wr_c, Wh_c], axis=1)
    cz = (bz_c @ Wz[:F_OUT] + bz).reshape(1, F_OUT)
    cr = (br_c @ Wr[:F_OUT] + br).reshape(1, F_OUT)
    ch = (bh_c @ Wh[:F_OUT] + bh).reshape(1, F_OUT)

    h = _tc_project(x, wcat)
    acc, dinv = _sc_aggregate(src2d, dst2d, w2d, h, n_rows, n_rows // NW)
    y, hn = _tc_gru(acc, h, dinv.reshape(N, 1), prev_hidden_state,
                    Wz[:F_OUT], Wz[F_OUT:], cz,
                    Wr[:F_OUT], Wr[F_OUT:], cr,
                    Wh[:F_OUT], Wh[F_OUT:], ch,
                    Wlin, blin.reshape(1, 1))
    return (y, hn)


# speedup vs baseline: 18.3801x; 1.1468x over previous
import functools

import jax
import jax.numpy as jnp
from jax import lax
from jax.experimental import pallas as pl
from jax.experimental.pallas import tpu as pltpu
from jax.experimental.pallas import tpu_sc as plsc

N = 10000
F_IN = 128
F_OUT = 32
F3 = 3 * F_OUT

NC = 2
NS = 16
NW = NC * NS

ROW = 128
RCH = 2
CE = ROW * RCH
DCH = 16
FV = F3 // 16

_SC_PARAMS = pltpu.CompilerParams(
    use_tc_tiling_on_sc=False, needs_layout_passes=False)


def _rsqrt_newton(d):
    i = plsc.bitcast(d, jnp.int32)
    y = plsc.bitcast(jnp.int32(0x5F3759DF) - (i >> 1), jnp.float32)
    for _ in range(3):
        y = y * (1.5 - 0.5 * d * y * y)
    return y


def _sc_degree(dst2d, w2d, n_rows):
    rows_per_core = n_rows // NC
    rows_per_tile = rows_per_core // NS
    n_chunks = rows_per_tile // DCH

    mesh = plsc.VectorSubcoreMesh(core_axis_name="c", subcore_axis_name="s")

    @functools.partial(
        pl.kernel,
        out_type=jax.ShapeDtypeStruct((NC, N), jnp.float32),
        mesh=mesh,
        scratch_types=[
            pltpu.VMEM((2, DCH, ROW), jnp.int32),
            pltpu.VMEM((2, DCH, ROW), jnp.float32),
            pltpu.VMEM((1000,), jnp.float32),
            pltpu.VMEM_SHARED((N,), jnp.float32),
            pltpu.SemaphoreType.DMA,
            pltpu.SemaphoreType.DMA,
        ],
        compiler_params=_SC_PARAMS,
    )
    def kern(dst_hbm, w_hbm, degp_out, dstb, wb, zl, deg_sh, sem_e, sem_s):
        c = lax.axis_index("c")
        s = lax.axis_index("s")
        z16 = jnp.zeros((16,), jnp.float32)

        @pl.loop(0, 1000 // 16)
        def _(i):
            zl[pl.ds(i * 16, 16)] = z16

        @pl.when(s < 10)
        def _():
            pltpu.sync_copy(zl, deg_sh.at[pl.ds(s * 1000, 1000)])

        plsc.subcore_barrier()

        base = c * rows_per_core + s * rows_per_tile

        def edge_dma(g, b, start):
            f = pltpu.async_copy if start else (
                lambda *a: pltpu.make_async_copy(*a).wait())
            f(dst_hbm.at[pl.ds(base + g * DCH, DCH)], dstb.at[b], sem_e)
            f(w_hbm.at[pl.ds(base + g * DCH, DCH)], wb.at[b], sem_e)

        edge_dma(0, 0, True)
        for g in range(n_chunks):
            b = g % 2
            edge_dma(g, b, False)
            if g >= 1:
                for j in range(DCH):
                    pltpu.make_async_copy(
                        wb.at[1 - b].at[j],
                        deg_sh.at[dstb.at[1 - b].at[j]], sem_s).wait()
            if g + 1 < n_chunks:
                edge_dma(g + 1, 1 - b, True)
            for j in range(DCH):
                pltpu.async_copy(wb.at[b].at[j],
                                 deg_sh.at[dstb.at[b].at[j]], sem_s,
                                 add=True)
        bl = (n_chunks - 1) % 2
        for j in range(DCH):
            pltpu.make_async_copy(wb.at[bl].at[j],
                                  deg_sh.at[dstb.at[bl].at[j]], sem_s).wait()

        plsc.subcore_barrier()

        @pl.when(s < 10)
        def _():
            sl = pl.ds(s * 1000, 1000)
            pltpu.sync_copy(deg_sh.at[sl], degp_out.at[c].at[sl])

    return kern(dst2d, w2d)


def _sc_messages(src2d, dst2d, w2d, h, degp, n_rows):
    rows_per_worker = n_rows // NW
    n_chunks = rows_per_worker // RCH
    npt = N // NS

    mesh = plsc.VectorSubcoreMesh(core_axis_name="c", subcore_axis_name="s")

    @functools.partial(
        pl.kernel,
        out_type=[
            jax.ShapeDtypeStruct((NC, N, F3), jnp.float32),
            jax.ShapeDtypeStruct((N,), jnp.float32),
        ],
        mesh=mesh,
        scratch_types=[
            pltpu.VMEM((N,), jnp.float32),
            pltpu.VMEM((2000,), jnp.float32),
            pltpu.VMEM((2000,), jnp.float32),
            pltpu.VMEM((2, RCH, ROW), jnp.int32),
            pltpu.VMEM((2, RCH, ROW), jnp.int32),
            pltpu.VMEM((2, RCH, ROW), jnp.float32),
            pltpu.VMEM((2, CE, F3), jnp.float32),
            pltpu.VMEM_SHARED((N, F3), jnp.float32),
            pltpu.SemaphoreType.DMA,
            pltpu.SemaphoreType.DMA,
            pltpu.SemaphoreType.DMA,
            pltpu.SemaphoreType.DMA,
        ],
        compiler_params=_SC_PARAMS,
    )
    def kern(src_hbm, dst_hbm, w_hbm, h_hbm, degp_hbm, acc_out, dinv_out,
             dinv_l, tp0, tp1, srcb, dstb, wb, rows, acc_sh,
             sem_e, sem_g, sem_sa, sem_sb):
        c = lax.axis_index("c")
        s = lax.axis_index("s")
        sem_s = (sem_sa, sem_sb)
        z16 = jnp.zeros((16,), jnp.float32)

        @pl.loop(0, CE)
        def _(e):
            for f in range(FV):
                rows[0, e, pl.ds(f * 16, 16)] = z16

        cur = 0
        while cur < npt:
            step = min(CE, npt - cur)
            pltpu.sync_copy(rows.at[0].at[pl.ds(0, step)],
                            acc_sh.at[pl.ds(s * npt + cur, step)])
            cur += step

        for t in range(N // 2000):
            sl = pl.ds(t * 2000, 2000)
            pltpu.sync_copy(degp_hbm.at[0].at[sl], tp0)
            pltpu.sync_copy(degp_hbm.at[1].at[sl], tp1)

            @pl.loop(0, 2000 // 16)
            def _(i):
                si = pl.ds(i * 16, 16)
                dinv_l[pl.ds(t * 2000 + i * 16, 16)] = _rsqrt_newton(
                    tp0[si] + tp1[si] + 1.0)

        @pl.when(jnp.logical_and(c == 0, s < 10))
        def _():
            sl = pl.ds(s * 1000, 1000)
            pltpu.sync_copy(dinv_l.at[sl], dinv_out.at[sl])

        plsc.subcore_barrier()

        w0row = (c * NS + s) * rows_per_worker

        def edge_dma(k, b, start):
            f = pltpu.async_copy if start else (
                lambda *a: pltpu.make_async_copy(*a).wait())
            r0 = pl.ds(w0row + k * RCH, RCH)
            f(src_hbm.at[r0], srcb.at[b], sem_e)
            f(dst_hbm.at[r0], dstb.at[b], sem_e)
            f(w_hbm.at[r0], wb.at[b], sem_e)

        def scatter(k_b, start):
            b, sem = k_b, sem_s[k_b]
            for j in range(RCH):
                srcr = rows.at[b].at[pl.ds(j * ROW, ROW)]
                dstr = acc_sh.at[dstb.at[b].at[j]]
                if start:
                    pltpu.async_copy(srcr, dstr, sem, add=True)
                else:
                    pltpu.make_async_copy(srcr, dstr, sem).wait()

        edge_dma(0, 0, True)

        @pl.loop(0, n_chunks // 2)
        def _(g):
            for b in range(2):
                k = g * 2 + b
                edge_dma(k, b, False)
                for j in range(RCH):
                    pltpu.async_copy(h_hbm.at[srcb.at[b].at[j]],
                                     rows.at[b].at[pl.ds(j * ROW, ROW)],
                                     sem_g)
                for j in range(RCH):
                    pltpu.make_async_copy(h_hbm.at[srcb.at[b].at[j]],
                                          rows.at[b].at[pl.ds(j * ROW, ROW)],
                                          sem_g).wait()

                @pl.loop(0, RCH)
                def _(r):
                    @pl.loop(0, ROW // 16)
                    def _(i):
                        idx = srcb[b, r, pl.ds(i * 16, 16)]
                        dv = plsc.load_gather(dinv_l, [idx])
                        av = wb[b, r, pl.ds(i * 16, 16)] * dv
                        e0 = r * ROW + i * 16
                        for j in range(16):
                            a = av[j]
                            for f in range(FV):
                                sf = pl.ds(f * 16, 16)
                                rows[b, e0 + j, sf] = rows[b, e0 + j, sf] * a

                scatter(b, True)

                @pl.when(k >= 1)
                def _():
                    scatter(1 - b, False)

                @pl.when(k + 1 < n_chunks)
                def _():
                    edge_dma(k + 1, 1 - b, True)

        scatter((n_chunks - 1) % 2, False)
        plsc.subcore_barrier()

        sl = pl.ds(s * npt, npt)
        pltpu.sync_copy(acc_sh.at[sl], acc_out.at[c].at[sl])

    return kern(src2d, dst2d, w2d, h, degp)


def _tc_project(x, wcat):
    nb = 5
    bs = N // nb

    def body(x_ref, w_ref, o_ref):
        o_ref[...] = jnp.dot(x_ref[...], w_ref[...],
                             preferred_element_type=jnp.float32)

    return pl.pallas_call(
        body,
        grid=(nb,),
        in_specs=[
            pl.BlockSpec((bs, F_IN), lambda i: (i, 0)),
            pl.BlockSpec((F_IN, F3), lambda i: (0, 0)),
        ],
        out_specs=pl.BlockSpec((bs, F3), lambda i: (i, 0)),
        out_shape=jax.ShapeDtypeStruct((N, F3), jnp.float32),
    )(x, wcat)


def _tc_gru(acc, h, dinv, hprev, wza, wzb, cz, wra, wrb, cr, wha, whb, ch,
            wlin, blin):
    nb = 5
    bs = N // nb

    def body(a0_ref, a1_ref, h_ref, di_ref, hp_ref, wza_ref, wzb_ref, cz_ref,
             wra_ref, wrb_ref, cr_ref, wha_ref, whb_ref, ch_ref,
             wlin_ref, blin_ref, y_ref, hn_ref):
        di = di_ref[...]
        hp = hp_ref[...]
        agg = (a0_ref[...] + a1_ref[...] + h_ref[...] * di) * di
        gz = agg[:, :F_OUT]
        gr = agg[:, F_OUT:2 * F_OUT]
        gh = agg[:, 2 * F_OUT:]
        f32 = jnp.float32
        z = jax.nn.sigmoid(jnp.dot(gz, wza_ref[...], preferred_element_type=f32)
                           + jnp.dot(hp, wzb_ref[...], preferred_element_type=f32)
                           + cz_ref[...])
        r = jax.nn.sigmoid(jnp.dot(gr, wra_ref[...], preferred_element_type=f32)
                           + jnp.dot(hp, wrb_ref[...], preferred_element_type=f32)
                           + cr_ref[...])
        ht = jnp.tanh(jnp.dot(gh, wha_ref[...], preferred_element_type=f32)
                      + jnp.dot(hp * r, whb_ref[...], preferred_element_type=f32)
                      + ch_ref[...])
        hn = z * hp + (1.0 - z) * ht
        hn_ref[...] = hn
        y_ref[...] = (jnp.dot(jax.nn.relu(hn), wlin_ref[...],
                              preferred_element_type=f32) + blin_ref[...])

    full = lambda r, c: pl.BlockSpec((r, c), lambda i: (0, 0))
    blk = lambda cdim: pl.BlockSpec((bs, cdim), lambda i: (i, 0))
    return pl.pallas_call(
        body,
        grid=(nb,),
        in_specs=[
            blk(F3), blk(F3), blk(F3), blk(1), blk(F_OUT),
            full(F_OUT, F_OUT), full(F_OUT, F_OUT), full(1, F_OUT),
            full(F_OUT, F_OUT), full(F_OUT, F_OUT), full(1, F_OUT),
            full(F_OUT, F_OUT), full(F_OUT, F_OUT), full(1, F_OUT),
            full(F_OUT, 1), full(1, 1),
        ],
        out_specs=[blk(1), blk(F_OUT)],
        out_shape=[
            jax.ShapeDtypeStruct((N, 1), jnp.float32),
            jax.ShapeDtypeStruct((N, F_OUT), jnp.float32),
        ],
    )(acc[0], acc[1], h, dinv, hprev, wza, wzb, cz, wra, wrb, cr,
      wha, whb, ch, wlin, blin)


def kernel(x, edge_index, edge_weight, prev_hidden_state, c,
           Wz_c, bz_c, Wr_c, br_c, Wh_c, bh_c,
           Wz, bz, Wr, br, Wh, bh, Wlin, blin):
    src, dst = edge_index[0], edge_index[1]
    e = src.shape[0]

    grain = NW * CE * 4
    epad = -(-e // grain) * grain
    pad = epad - e
    if pad:
        src = jnp.concatenate([src, jnp.zeros((pad,), src.dtype)])
        dst = jnp.concatenate([dst, jnp.zeros((pad,), dst.dtype)])
        edge_weight = jnp.concatenate(
            [edge_weight, jnp.zeros((pad,), edge_weight.dtype)])
    n_rows = epad // ROW
    src2d = src.reshape(n_rows, ROW)
    dst2d = dst.reshape(n_rows, ROW)
    w2d = edge_weight.reshape(n_rows, ROW)

    wcat = jnp.concatenate([Wz_c, Wr_c, Wh_c], axis=1)
    cz = (bz_c @ Wz[:F_OUT] + bz).reshape(1, F_OUT)
    cr = (br_c @ Wr[:F_OUT] + br).reshape(1, F_OUT)
    ch = (bh_c @ Wh[:F_OUT] + bh).reshape(1, F_OUT)

    degp = _sc_degree(dst2d, w2d, n_rows)
    h = _tc_project(x, wcat)
    acc, dinv = _sc_messages(src2d, dst2d, w2d, h, degp, n_rows)
    y, hn = _tc_gru(acc, h, dinv.reshape(N, 1), prev_hidden_state,
                    Wz[:F_OUT], Wz[F_OUT:], cz,
                    Wr[:F_OUT], Wr[F_OUT:], cr,
                    Wh[:F_OUT], Wh[F_OUT:], ch,
                    Wlin, blin.reshape(1, 1))
    return (y, hn)


# speedup vs baseline: 26.6818x; 1.4517x over previous
import functools

import jax
import jax.numpy as jnp
from jax import lax
from jax.experimental import pallas as pl
from jax.experimental.pallas import tpu as pltpu
from jax.experimental.pallas import tpu_sc as plsc

N = 10000
F_IN = 128
F_OUT = 32
F3 = 3 * F_OUT

NC = 2
NS = 16
NW = NC * NS

ROW = 128
RCH = 2
CE = ROW * RCH
DCH = 16
FV = F3 // 16

_SC_PARAMS = pltpu.CompilerParams(
    use_tc_tiling_on_sc=False, needs_layout_passes=False)


def _rsqrt_newton(d):
    i = plsc.bitcast(d, jnp.int32)
    y = plsc.bitcast(jnp.int32(0x5F3759DF) - (i >> 1), jnp.float32)
    for _ in range(3):
        y = y * (1.5 - 0.5 * d * y * y)
    return y


def _sc_degree(dst2d, w2d, n_rows):
    rows_per_core = n_rows // NC
    rows_per_tile = rows_per_core // NS
    n_chunks = rows_per_tile // DCH

    mesh = plsc.VectorSubcoreMesh(core_axis_name="c", subcore_axis_name="s")

    @functools.partial(
        pl.kernel,
        out_type=jax.ShapeDtypeStruct((NC, N), jnp.float32),
        mesh=mesh,
        scratch_types=[
            pltpu.VMEM((2, DCH, ROW), jnp.int32),
            pltpu.VMEM((2, DCH, ROW), jnp.float32),
            pltpu.VMEM((1000,), jnp.float32),
            pltpu.VMEM_SHARED((N,), jnp.float32),
            pltpu.SemaphoreType.DMA,
            pltpu.SemaphoreType.DMA,
        ],
        compiler_params=_SC_PARAMS,
    )
    def kern(dst_hbm, w_hbm, degp_out, dstb, wb, zl, deg_sh, sem_e, sem_s):
        c = lax.axis_index("c")
        s = lax.axis_index("s")
        z16 = jnp.zeros((16,), jnp.float32)

        @pl.loop(0, 1000 // 16)
        def _(i):
            zl[pl.ds(i * 16, 16)] = z16

        @pl.when(s < 10)
        def _():
            pltpu.sync_copy(zl, deg_sh.at[pl.ds(s * 1000, 1000)])

        plsc.subcore_barrier()

        base = c * rows_per_core + s * rows_per_tile

        def edge_dma(g, b, start):
            f = pltpu.async_copy if start else (
                lambda *a: pltpu.make_async_copy(*a).wait())
            f(dst_hbm.at[pl.ds(base + g * DCH, DCH)], dstb.at[b], sem_e)
            f(w_hbm.at[pl.ds(base + g * DCH, DCH)], wb.at[b], sem_e)

        edge_dma(0, 0, True)
        for g in range(n_chunks):
            b = g % 2
            edge_dma(g, b, False)
            if g >= 1:
                for j in range(DCH):
                    pltpu.make_async_copy(
                        wb.at[1 - b].at[j],
                        deg_sh.at[dstb.at[1 - b].at[j]], sem_s).wait()
            if g + 1 < n_chunks:
                edge_dma(g + 1, 1 - b, True)
            for j in range(DCH):
                pltpu.async_copy(wb.at[b].at[j],
                                 deg_sh.at[dstb.at[b].at[j]], sem_s,
                                 add=True)
        bl = (n_chunks - 1) % 2
        for j in range(DCH):
            pltpu.make_async_copy(wb.at[bl].at[j],
                                  deg_sh.at[dstb.at[bl].at[j]], sem_s).wait()

        plsc.subcore_barrier()

        @pl.when(s < 10)
        def _():
            sl = pl.ds(s * 1000, 1000)
            pltpu.sync_copy(deg_sh.at[sl], degp_out.at[c].at[sl])

    return kern(dst2d, w2d)


def _sc_messages(src2d, dst2d, w2d, h, degp, n_rows):
    pair_rows = n_rows // NS
    rw0 = int(round(pair_rows * 0.65 / (2 * RCH))) * 2 * RCH
    rw0 = min(max(rw0, 2 * RCH), pair_rows - 2 * RCH)
    rw1 = pair_rows - rw0
    nch0, nch1 = rw0 // RCH, rw1 // RCH
    npt = N // NS

    mesh = plsc.VectorSubcoreMesh(core_axis_name="c", subcore_axis_name="s")

    @functools.partial(
        pl.kernel,
        out_type=[
            jax.ShapeDtypeStruct((NC, N, F3), jnp.float32),
            jax.ShapeDtypeStruct((N,), jnp.float32),
        ],
        mesh=mesh,
        scratch_types=[
            pltpu.VMEM((N,), jnp.float32),
            pltpu.VMEM((2000,), jnp.float32),
            pltpu.VMEM((2000,), jnp.float32),
            pltpu.VMEM((2, RCH, ROW), jnp.int32),
            pltpu.VMEM((2, RCH, ROW), jnp.int32),
            pltpu.VMEM((2, RCH, ROW), jnp.float32),
            pltpu.VMEM((2, CE, F3), jnp.float32),
            pltpu.VMEM_SHARED((N, F3), jnp.float32),
            pltpu.SemaphoreType.DMA,
            pltpu.SemaphoreType.DMA,
            pltpu.SemaphoreType.DMA,
            pltpu.SemaphoreType.DMA,
        ],
        compiler_params=_SC_PARAMS,
    )
    def kern(src_hbm, dst_hbm, w_hbm, h_hbm, degp_hbm, acc_out, dinv_out,
             dinv_l, tp0, tp1, srcb, dstb, wb, rows, acc_sh,
             sem_e, sem_g, sem_sa, sem_sb):
        c = lax.axis_index("c")
        s = lax.axis_index("s")
        sem_s = (sem_sa, sem_sb)
        z16 = jnp.zeros((16,), jnp.float32)

        @pl.loop(0, CE)
        def _(e):
            for f in range(FV):
                rows[0, e, pl.ds(f * 16, 16)] = z16

        cur = 0
        while cur < npt:
            step = min(CE, npt - cur)
            pltpu.sync_copy(rows.at[0].at[pl.ds(0, step)],
                            acc_sh.at[pl.ds(s * npt + cur, step)])
            cur += step

        for t in range(N // 2000):
            sl = pl.ds(t * 2000, 2000)
            pltpu.sync_copy(degp_hbm.at[0].at[sl], tp0)
            pltpu.sync_copy(degp_hbm.at[1].at[sl], tp1)

            @pl.loop(0, 2000 // 16)
            def _(i):
                si = pl.ds(i * 16, 16)
                dinv_l[pl.ds(t * 2000 + i * 16, 16)] = _rsqrt_newton(
                    tp0[si] + tp1[si] + 1.0)

        @pl.when(jnp.logical_and(c == 0, s < 10))
        def _():
            sl = pl.ds(s * 1000, 1000)
            pltpu.sync_copy(dinv_l.at[sl], dinv_out.at[sl])

        plsc.subcore_barrier()

        n_chunks = jnp.where(c == 0, nch0, nch1)
        w0row = jnp.where(c == 0, s * rw0, NS * rw0 + s * rw1)

        def edge_dma(k, b, start):
            f = pltpu.async_copy if start else (
                lambda *a: pltpu.make_async_copy(*a).wait())
            r0 = pl.ds(w0row + k * RCH, RCH)
            f(src_hbm.at[r0], srcb.at[b], sem_e)
            f(dst_hbm.at[r0], dstb.at[b], sem_e)
            f(w_hbm.at[r0], wb.at[b], sem_e)

        def gather(k_b, start):
            b = k_b
            for j in range(RCH):
                srcr = h_hbm.at[srcb.at[b].at[j]]
                dstr = rows.at[b].at[pl.ds(j * ROW, ROW)]
                if start:
                    pltpu.async_copy(srcr, dstr, sem_g)
                else:
                    pltpu.make_async_copy(srcr, dstr, sem_g).wait()

        def scatter(k_b, start):
            b, sem = k_b, sem_s[k_b]
            for j in range(RCH):
                srcr = rows.at[b].at[pl.ds(j * ROW, ROW)]
                dstr = acc_sh.at[dstb.at[b].at[j]]
                if start:
                    pltpu.async_copy(srcr, dstr, sem, add=True)
                else:
                    pltpu.make_async_copy(srcr, dstr, sem).wait()

        edge_dma(0, 0, True)
        edge_dma(0, 0, False)
        gather(0, True)

        @pl.loop(0, (nch0 + 1) // 2)
        def _(g):
            for b in range(2):
                k = g * 2 + b

                @pl.when(k < n_chunks)
                def _():
                    @pl.when(k >= 1)
                    def _():
                        scatter(1 - b, False)

                    @pl.when(k + 1 < n_chunks)
                    def _():
                        edge_dma(k + 1, 1 - b, True)

                    gather(b, False)

                    @pl.when(k + 1 < n_chunks)
                    def _():
                        edge_dma(k + 1, 1 - b, False)
                        gather(1 - b, True)

                    @pl.loop(0, RCH)
                    def _(r):
                        @pl.loop(0, ROW // 16)
                        def _(i):
                            idx = srcb[b, r, pl.ds(i * 16, 16)]
                            dv = plsc.load_gather(dinv_l, [idx])
                            av = wb[b, r, pl.ds(i * 16, 16)] * dv
                            e0 = r * ROW + i * 16
                            for j in range(16):
                                a = av[j]
                                for f in range(FV):
                                    sf = pl.ds(f * 16, 16)
                                    rows[b, e0 + j, sf] = \
                                        rows[b, e0 + j, sf] * a

                    scatter(b, True)

        scatter(1, False)
        plsc.subcore_barrier()

        sl = pl.ds(s * npt, npt)
        pltpu.sync_copy(acc_sh.at[sl], acc_out.at[c].at[sl])

    return kern(src2d, dst2d, w2d, h, degp)


def _tc_project(x, wcat):
    nb = 5
    bs = N // nb

    def body(x_ref, w_ref, o_ref):
        o_ref[...] = jnp.dot(x_ref[...], w_ref[...],
                             preferred_element_type=jnp.float32)

    return pl.pallas_call(
        body,
        grid=(nb,),
        in_specs=[
            pl.BlockSpec((bs, F_IN), lambda i: (i, 0)),
            pl.BlockSpec((F_IN, F3), lambda i: (0, 0)),
        ],
        out_specs=pl.BlockSpec((bs, F3), lambda i: (i, 0)),
        out_shape=jax.ShapeDtypeStruct((N, F3), jnp.float32),
    )(x, wcat)


def _tc_gru(acc, h, dinv, hprev, wza, wzb, cz, wra, wrb, cr, wha, whb, ch,
            wlin, blin):
    nb = 5
    bs = N // nb

    def body(a0_ref, a1_ref, h_ref, di_ref, hp_ref, wza_ref, wzb_ref, cz_ref,
             wra_ref, wrb_ref, cr_ref, wha_ref, whb_ref, ch_ref,
             wlin_ref, blin_ref, y_ref, hn_ref):
        di = di_ref[...]
        hp = hp_ref[...]
        agg = (a0_ref[...] + a1_ref[...] + h_ref[...] * di) * di
        gz = agg[:, :F_OUT]
        gr = agg[:, F_OUT:2 * F_OUT]
        gh = agg[:, 2 * F_OUT:]
        f32 = jnp.float32
        z = jax.nn.sigmoid(jnp.dot(gz, wza_ref[...], preferred_element_type=f32)
                           + jnp.dot(hp, wzb_ref[...], preferred_element_type=f32)
                           + cz_ref[...])
        r = jax.nn.sigmoid(jnp.dot(gr, wra_ref[...], preferred_element_type=f32)
                           + jnp.dot(hp, wrb_ref[...], preferred_element_type=f32)
                           + cr_ref[...])
        ht = jnp.tanh(jnp.dot(gh, wha_ref[...], preferred_element_type=f32)
                      + jnp.dot(hp * r, whb_ref[...], preferred_element_type=f32)
                      + ch_ref[...])
        hn = z * hp + (1.0 - z) * ht
        hn_ref[...] = hn
        y_ref[...] = (jnp.dot(jax.nn.relu(hn), wlin_ref[...],
                              preferred_element_type=f32) + blin_ref[...])

    full = lambda r, c: pl.BlockSpec((r, c), lambda i: (0, 0))
    blk = lambda cdim: pl.BlockSpec((bs, cdim), lambda i: (i, 0))
    return pl.pallas_call(
        body,
        grid=(nb,),
        in_specs=[
            blk(F3), blk(F3), blk(F3), blk(1), blk(F_OUT),
            full(F_OUT, F_OUT), full(F_OUT, F_OUT), full(1, F_OUT),
            full(F_OUT, F_OUT), full(F_OUT, F_OUT), full(1, F_OUT),
            full(F_OUT, F_OUT), full(F_OUT, F_OUT), full(1, F_OUT),
            full(F_OUT, 1), full(1, 1),
        ],
        out_specs=[blk(1), blk(F_OUT)],
        out_shape=[
            jax.ShapeDtypeStruct((N, 1), jnp.float32),
            jax.ShapeDtypeStruct((N, F_OUT), jnp.float32),
        ],
    )(acc[0], acc[1], h, dinv, hprev, wza, wzb, cz, wra, wrb, cr,
      wha, whb, ch, wlin, blin)


def kernel(x, edge_index, edge_weight, prev_hidden_state, c,
           Wz_c, bz_c, Wr_c, br_c, Wh_c, bh_c,
           Wz, bz, Wr, br, Wh, bh, Wlin, blin):
    src, dst = edge_index[0], edge_index[1]
    e = src.shape[0]

    grain = NW * CE * 4
    epad = -(-e // grain) * grain
    pad = epad - e
    if pad:
        src = jnp.concatenate([src, jnp.zeros((pad,), src.dtype)])
        dst = jnp.concatenate([dst, jnp.zeros((pad,), dst.dtype)])
        edge_weight = jnp.concatenate(
            [edge_weight, jnp.zeros((pad,), edge_weight.dtype)])
    n_rows = epad // ROW
    src2d = src.reshape(n_rows, ROW)
    dst2d = dst.reshape(n_rows, ROW)
    w2d = edge_weight.reshape(n_rows, ROW)

    wcat = jnp.concatenate([Wz_c, Wr_c, Wh_c], axis=1)
    cz = (bz_c @ Wz[:F_OUT] + bz).reshape(1, F_OUT)
    cr = (br_c @ Wr[:F_OUT] + br).reshape(1, F_OUT)
    ch = (bh_c @ Wh[:F_OUT] + bh).reshape(1, F_OUT)

    degp = _sc_degree(dst2d, w2d, n_rows)
    h = _tc_project(x, wcat)
    acc, dinv = _sc_messages(src2d, dst2d, w2d, h, degp, n_rows)
    y, hn = _tc_gru(acc, h, dinv.reshape(N, 1), prev_hidden_state,
                    Wz[:F_OUT], Wz[F_OUT:], cz,
                    Wr[:F_OUT], Wr[F_OUT:], cr,
                    Wh[:F_OUT], Wh[F_OUT:], ch,
                    Wlin, blin.reshape(1, 1))
    return (y, hn)
